# baseline (speedup 1.0000x reference)
import functools

import jax
import jax.numpy as jnp
from jax import lax
from jax.experimental import pallas as pl
from jax.experimental.pallas import tpu as pltpu
from jax.experimental.pallas import tpu_sc as plsc

N = 10000
E = 320000
D = 128

NC = 2
NS = 16
NW = NC * NS
PER_W = E // NW
CH = 100
NCHUNK = PER_W // CH
NPH = 4
HCH = NCHUNK // NPH
NSLOT = 3
RPT = 624
TAIL = N - NS * RPT
TAIL_OFF = NS * RPT

_mesh = plsc.VectorSubcoreMesh(core_axis_name="c", subcore_axis_name="s")


@functools.partial(
    pl.kernel,
    out_type=jax.ShapeDtypeStruct((NC, N, D), jnp.float32),
    mesh=_mesh,
    scratch_types=[
        pltpu.VMEM((HCH, CH), jnp.int32),
        pltpu.VMEM((HCH, CH), jnp.int32),
        pltpu.VMEM((NSLOT * CH, D), jnp.float32),
        pltpu.VMEM_SHARED((N, D), jnp.float32),
        pltpu.SemaphoreType.DMA,
        pltpu.SemaphoreType.DMA,
    ],
)
def _sc_aggregate(m_hbm, src_hbm, dst_hbm, zero_hbm, part_hbm,
                  src_v, dst_v, rows_v, acc_sh, sem_g, sem_s):
    c = lax.axis_index("c")
    s = lax.axis_index("s")
    wid = c * NS + s
    pltpu.sync_copy(zero_hbm.at[pl.ds(s * RPT, RPT)],
                    acc_sh.at[pl.ds(s * RPT, RPT)])

    @pl.when(s == NS - 1)
    def _():
        pltpu.sync_copy(zero_hbm.at[pl.ds(TAIL_OFF, TAIL)],
                        acc_sh.at[pl.ds(TAIL_OFF, TAIL)])
    plsc.subcore_barrier()

    for ph in range(NPH):
        pltpu.sync_copy(src_hbm.at[wid, ph], src_v)
        pltpu.sync_copy(dst_hbm.at[wid, ph], dst_v)
        pltpu.async_copy(m_hbm.at[src_v.at[0]], rows_v.at[pl.ds(0, CH)],
                         sem_g)

        def body(j, carry):
            slot = lax.rem(j, NSLOT) * CH
            pltpu.make_async_copy(m_hbm.at[src_v.at[j]],
                                  rows_v.at[pl.ds(slot, CH)], sem_g).wait()

            @pl.when(j < HCH - 1)
            def _():
                nslot = lax.rem(j + 1, NSLOT) * CH

                @pl.when(j >= NSLOT - 1)
                def _():
                    pltpu.make_async_copy(
                        rows_v.at[pl.ds(nslot, CH)],
                        acc_sh.at[dst_v.at[j + 1 - NSLOT]], sem_s).wait()

                pltpu.async_copy(m_hbm.at[src_v.at[j + 1]],
                                 rows_v.at[pl.ds(nslot, CH)], sem_g)

            pltpu.async_copy(rows_v.at[pl.ds(slot, CH)],
                             acc_sh.at[dst_v.at[j]], sem_s, add=True)
            return carry

        lax.fori_loop(0, HCH, body, 0)
        for t in range(NSLOT):
            pj = HCH - NSLOT + t
            pltpu.make_async_copy(
                rows_v.at[pl.ds((pj % NSLOT) * CH, CH)],
                acc_sh.at[dst_v.at[pj]], sem_s).wait()
    plsc.subcore_barrier()
    pltpu.sync_copy(acc_sh.at[pl.ds(s * RPT, RPT)],
                    part_hbm.at[c, pl.ds(s * RPT, RPT)])

    @pl.when(s == NS - 1)
    def _():
        pltpu.sync_copy(acc_sh.at[pl.ds(TAIL_OFF, TAIL)],
                        part_hbm.at[c, pl.ds(TAIL_OFF, TAIL)])


_BLK = 1000


def _mm1_body(x_ref, w_ref, o_ref):
    o_ref[...] = jnp.dot(x_ref[...], w_ref[...],
                         preferred_element_type=jnp.float32)


def _mid_body(p_ref, b_ref, w_ref, o_ref):
    h = jnp.maximum(p_ref[0] + p_ref[1] + b_ref[...], 0.0)
    o_ref[...] = jnp.dot(h, w_ref[...], preferred_element_type=jnp.float32)


def _fin_body(p_ref, b_ref, o_ref):
    o_ref[...] = p_ref[0] + p_ref[1] + b_ref[...]


_mm1 = pl.pallas_call(
    _mm1_body,
    grid=(N // _BLK,),
    in_specs=[
        pl.BlockSpec((_BLK, D), lambda i: (i, 0)),
        pl.BlockSpec((D, D), lambda i: (0, 0)),
    ],
    out_specs=pl.BlockSpec((_BLK, D), lambda i: (i, 0)),
    out_shape=jax.ShapeDtypeStruct((N, D), jnp.float32),
)

_mid = pl.pallas_call(
    _mid_body,
    grid=(N // _BLK,),
    in_specs=[
        pl.BlockSpec((NC, _BLK, D), lambda i: (0, i, 0)),
        pl.BlockSpec((1, D), lambda i: (0, 0)),
        pl.BlockSpec((D, D), lambda i: (0, 0)),
    ],
    out_specs=pl.BlockSpec((_BLK, D), lambda i: (i, 0)),
    out_shape=jax.ShapeDtypeStruct((N, D), jnp.float32),
)

_fin = pl.pallas_call(
    _fin_body,
    grid=(N // _BLK,),
    in_specs=[
        pl.BlockSpec((NC, _BLK, D), lambda i: (0, i, 0)),
        pl.BlockSpec((1, D), lambda i: (0, 0)),
    ],
    out_specs=pl.BlockSpec((_BLK, D), lambda i: (i, 0)),
    out_shape=jax.ShapeDtypeStruct((N, D), jnp.float32),
)


def kernel(x, edge_index, W1, b1, W2, b2):
    src = edge_index[0].astype(jnp.int32).reshape(NW, NPH, HCH, CH)
    dst = edge_index[1].astype(jnp.int32).reshape(NW, NPH, HCH, CH)
    zeros = jnp.zeros((N, D), jnp.float32)
    b1r = b1.reshape(1, D)
    b2r = b2.reshape(1, D)

    m1 = _mm1(x, W1)
    p1 = _sc_aggregate(m1, src, dst, zeros)
    m2 = _mid(p1, b1r, W2)
    p2 = _sc_aggregate(m2, src, dst, zeros)
    return _fin(p2, b2r)


# speedup vs baseline: 1.0879x; 1.0879x over previous
import functools

import jax
import jax.numpy as jnp
from jax import lax
from jax.experimental import pallas as pl
from jax.experimental.pallas import tpu as pltpu
from jax.experimental.pallas import tpu_sc as plsc

N = 10000
E = 320000
D = 128

NC = 2
NS = 16
NW = NC * NS
PER_W = E // NW
CH = 125
NCHUNK = PER_W // CH
NPH = 2
HCH = NCHUNK // NPH
NSLOT = 2
RPT = 624
TAIL = N - NS * RPT
TAIL_OFF = NS * RPT

_mesh = plsc.VectorSubcoreMesh(core_axis_name="c", subcore_axis_name="s")


@functools.partial(
    pl.kernel,
    out_type=jax.ShapeDtypeStruct((NC, N, D), jnp.float32),
    mesh=_mesh,
    scratch_types=[
        pltpu.VMEM((HCH, CH), jnp.int32),
        pltpu.VMEM((HCH, CH), jnp.int32),
        pltpu.VMEM((NSLOT * CH, D), jnp.float32),
        pltpu.VMEM_SHARED((N, D), jnp.float32),
        pltpu.SemaphoreType.DMA,
        pltpu.SemaphoreType.DMA,
    ],
)
def _sc_aggregate(m_hbm, src_hbm, dst_hbm, zero_hbm, part_hbm,
                  src_v, dst_v, rows_v, acc_sh, sem_g, sem_s):
    c = lax.axis_index("c")
    s = lax.axis_index("s")
    wid = c * NS + s
    pltpu.sync_copy(zero_hbm.at[pl.ds(s * RPT, RPT)],
                    acc_sh.at[pl.ds(s * RPT, RPT)])

    @pl.when(s == NS - 1)
    def _():
        pltpu.sync_copy(zero_hbm.at[pl.ds(TAIL_OFF, TAIL)],
                        acc_sh.at[pl.ds(TAIL_OFF, TAIL)])
    plsc.subcore_barrier()

    for ph in range(NPH):
        pltpu.sync_copy(src_hbm.at[wid, ph], src_v)
        pltpu.sync_copy(dst_hbm.at[wid, ph], dst_v)
        pltpu.async_copy(m_hbm.at[src_v.at[0]], rows_v.at[pl.ds(0, CH)],
                         sem_g)

        def body(j, carry):
            slot = lax.rem(j, NSLOT) * CH
            pltpu.make_async_copy(m_hbm.at[src_v.at[j]],
                                  rows_v.at[pl.ds(slot, CH)], sem_g).wait()

            @pl.when(j < HCH - 1)
            def _():
                nslot = lax.rem(j + 1, NSLOT) * CH

                @pl.when(j >= NSLOT - 1)
                def _():
                    pltpu.make_async_copy(
                        rows_v.at[pl.ds(nslot, CH)],
                        acc_sh.at[dst_v.at[j + 1 - NSLOT]], sem_s).wait()

                pltpu.async_copy(m_hbm.at[src_v.at[j + 1]],
                                 rows_v.at[pl.ds(nslot, CH)], sem_g)

            pltpu.async_copy(rows_v.at[pl.ds(slot, CH)],
                             acc_sh.at[dst_v.at[j]], sem_s, add=True)
            return carry

        lax.fori_loop(0, HCH, body, 0)
        for t in range(NSLOT):
            pj = HCH - NSLOT + t
            pltpu.make_async_copy(
                rows_v.at[pl.ds((pj % NSLOT) * CH, CH)],
                acc_sh.at[dst_v.at[pj]], sem_s).wait()
    plsc.subcore_barrier()
    pltpu.sync_copy(acc_sh.at[pl.ds(s * RPT, RPT)],
                    part_hbm.at[c, pl.ds(s * RPT, RPT)])

    @pl.when(s == NS - 1)
    def _():
        pltpu.sync_copy(acc_sh.at[pl.ds(TAIL_OFF, TAIL)],
                        part_hbm.at[c, pl.ds(TAIL_OFF, TAIL)])


_BLK = 1000


def _mm1_body(x_ref, w_ref, o_ref):
    o_ref[...] = jnp.dot(x_ref[...], w_ref[...],
                         preferred_element_type=jnp.float32)


def _mid_body(p_ref, b_ref, w_ref, o_ref):
    h = jnp.maximum(p_ref[0] + p_ref[1] + b_ref[...], 0.0)
    o_ref[...] = jnp.dot(h, w_ref[...], preferred_element_type=jnp.float32)


def _fin_body(p_ref, b_ref, o_ref):
    o_ref[...] = p_ref[0] + p_ref[1] + b_ref[...]


_mm1 = pl.pallas_call(
    _mm1_body,
    grid=(N // _BLK,),
    in_specs=[
        pl.BlockSpec((_BLK, D), lambda i: (i, 0)),
        pl.BlockSpec((D, D), lambda i: (0, 0)),
    ],
    out_specs=pl.BlockSpec((_BLK, D), lambda i: (i, 0)),
    out_shape=jax.ShapeDtypeStruct((N, D), jnp.float32),
)

_mid = pl.pallas_call(
    _mid_body,
    grid=(N // _BLK,),
    in_specs=[
        pl.BlockSpec((NC, _BLK, D), lambda i: (0, i, 0)),
        pl.BlockSpec((1, D), lambda i: (0, 0)),
        pl.BlockSpec((D, D), lambda i: (0, 0)),
    ],
    out_specs=pl.BlockSpec((_BLK, D), lambda i: (i, 0)),
    out_shape=jax.ShapeDtypeStruct((N, D), jnp.float32),
)

_fin = pl.pallas_call(
    _fin_body,
    grid=(N // _BLK,),
    in_specs=[
        pl.BlockSpec((NC, _BLK, D), lambda i: (0, i, 0)),
        pl.BlockSpec((1, D), lambda i: (0, 0)),
    ],
    out_specs=pl.BlockSpec((_BLK, D), lambda i: (i, 0)),
    out_shape=jax.ShapeDtypeStruct((N, D), jnp.float32),
)


def kernel(x, edge_index, W1, b1, W2, b2):
    src = edge_index[0].astype(jnp.int32).reshape(NW, NPH, HCH, CH)
    dst = edge_index[1].astype(jnp.int32).reshape(NW, NPH, HCH, CH)
    zeros = jnp.zeros((N, D), jnp.float32)
    b1r = b1.reshape(1, D)
    b2r = b2.reshape(1, D)

    m1 = _mm1(x, W1)
    p1 = _sc_aggregate(m1, src, dst, zeros)
    m2 = _mid(p1, b1r, W2)
    p2 = _sc_aggregate(m2, src, dst, zeros)
    return _fin(p2, b2r)


# speedup vs baseline: 1.1163x; 1.0261x over previous
import functools

import jax
import jax.numpy as jnp
from jax import lax
from jax.experimental import pallas as pl
from jax.experimental.pallas import tpu as pltpu
from jax.experimental.pallas import tpu_sc as plsc

N = 10000
E = 320000
D = 128

NC = 2
NS = 16
NW = NC * NS
PER_W = E // NW
CH = 125
NCHUNK = PER_W // CH
NPH = 2
HCH = NCHUNK // NPH
NSLOT = 2
RPT = 624
TAIL = N - NS * RPT
TAIL_OFF = NS * RPT

_mesh = plsc.VectorSubcoreMesh(core_axis_name="c", subcore_axis_name="s")


@functools.partial(
    pl.kernel,
    out_type=jax.ShapeDtypeStruct((NC, N, D), jnp.float32),
    mesh=_mesh,
    scratch_types=[
        pltpu.VMEM((HCH, CH), jnp.int32),
        pltpu.VMEM((HCH, CH), jnp.int32),
        pltpu.VMEM((NSLOT * CH, D), jnp.float32),
        pltpu.VMEM_SHARED((N, D), jnp.float32),
        pltpu.SemaphoreType.DMA,
        pltpu.SemaphoreType.DMA,
    ],
)
def _sc_aggregate(m_hbm, src_hbm, dst_hbm, zero_hbm, part_hbm,
                  src_v, dst_v, rows_v, acc_sh, sem_g, sem_s):
    c = lax.axis_index("c")
    s = lax.axis_index("s")
    wid = c * NS + s
    pltpu.sync_copy(zero_hbm.at[pl.ds(s * RPT, RPT)],
                    acc_sh.at[pl.ds(s * RPT, RPT)])

    @pl.when(s == NS - 1)
    def _():
        pltpu.sync_copy(zero_hbm.at[pl.ds(TAIL_OFF, TAIL)],
                        acc_sh.at[pl.ds(TAIL_OFF, TAIL)])
    plsc.subcore_barrier()

    for ph in range(NPH):
        pltpu.sync_copy(src_hbm.at[wid, ph], src_v)
        pltpu.sync_copy(dst_hbm.at[wid, ph], dst_v)
        pltpu.async_copy(m_hbm.at[src_v.at[0]], rows_v.at[pl.ds(0, CH)],
                         sem_g)

        def body(j, carry):
            slot = lax.rem(j, NSLOT) * CH
            pltpu.make_async_copy(m_hbm.at[src_v.at[j]],
                                  rows_v.at[pl.ds(slot, CH)], sem_g).wait()

            @pl.when(j < HCH - 1)
            def _():
                nslot = lax.rem(j + 1, NSLOT) * CH

                @pl.when(j >= NSLOT - 1)
                def _():
                    pltpu.make_async_copy(
                        rows_v.at[pl.ds(nslot, CH)],
                        acc_sh.at[dst_v.at[j + 1 - NSLOT]], sem_s).wait()

                pltpu.async_copy(m_hbm.at[src_v.at[j + 1]],
                                 rows_v.at[pl.ds(nslot, CH)], sem_g)

            pltpu.async_copy(rows_v.at[pl.ds(slot, CH)],
                             acc_sh.at[dst_v.at[j]], sem_s, add=True)
            return carry

        lax.fori_loop(0, HCH, body, 0)
        for t in range(NSLOT):
            pj = HCH - NSLOT + t
            pltpu.make_async_copy(
                rows_v.at[pl.ds((pj % NSLOT) * CH, CH)],
                acc_sh.at[dst_v.at[pj]], sem_s).wait()
    plsc.subcore_barrier()
    pltpu.sync_copy(acc_sh.at[pl.ds(s * RPT, RPT)],
                    part_hbm.at[c, pl.ds(s * RPT, RPT)])

    @pl.when(s == NS - 1)
    def _():
        pltpu.sync_copy(acc_sh.at[pl.ds(TAIL_OFF, TAIL)],
                        part_hbm.at[c, pl.ds(TAIL_OFF, TAIL)])


_BLK = 2000


def _mm1_body(x_ref, w_ref, o_ref):
    o_ref[...] = jnp.dot(x_ref[...], w_ref[...],
                         preferred_element_type=jnp.float32)


def _mid_body(p_ref, b_ref, w_ref, o_ref):
    h = jnp.maximum(p_ref[0] + p_ref[1] + b_ref[...], 0.0)
    o_ref[...] = jnp.dot(h, w_ref[...], preferred_element_type=jnp.float32)


def _fin_body(p_ref, b_ref, o_ref):
    o_ref[...] = p_ref[0] + p_ref[1] + b_ref[...]


_mm1 = pl.pallas_call(
    _mm1_body,
    grid=(N // _BLK,),
    in_specs=[
        pl.BlockSpec((_BLK, D), lambda i: (i, 0)),
        pl.BlockSpec((D, D), lambda i: (0, 0)),
    ],
    out_specs=pl.BlockSpec((_BLK, D), lambda i: (i, 0)),
    out_shape=jax.ShapeDtypeStruct((N, D), jnp.float32),
)

_mid = pl.pallas_call(
    _mid_body,
    grid=(N // _BLK,),
    in_specs=[
        pl.BlockSpec((NC, _BLK, D), lambda i: (0, i, 0)),
        pl.BlockSpec((1, D), lambda i: (0, 0)),
        pl.BlockSpec((D, D), lambda i: (0, 0)),
    ],
    out_specs=pl.BlockSpec((_BLK, D), lambda i: (i, 0)),
    out_shape=jax.ShapeDtypeStruct((N, D), jnp.float32),
)

_fin = pl.pallas_call(
    _fin_body,
    grid=(N // _BLK,),
    in_specs=[
        pl.BlockSpec((NC, _BLK, D), lambda i: (0, i, 0)),
        pl.BlockSpec((1, D), lambda i: (0, 0)),
    ],
    out_specs=pl.BlockSpec((_BLK, D), lambda i: (i, 0)),
    out_shape=jax.ShapeDtypeStruct((N, D), jnp.float32),
)


def kernel(x, edge_index, W1, b1, W2, b2):
    src = edge_index[0].astype(jnp.int32).reshape(NW, NPH, HCH, CH)
    dst = edge_index[1].astype(jnp.int32).reshape(NW, NPH, HCH, CH)
    zeros = jnp.zeros((N, D), jnp.float32)
    b1r = b1.reshape(1, D)
    b2r = b2.reshape(1, D)

    m1 = _mm1(x, W1)
    p1 = _sc_aggregate(m1, src, dst, zeros)
    m2 = _mid(p1, b1r, W2)
    p2 = _sc_aggregate(m2, src, dst, zeros)
    return _fin(p2, b2r)


# speedup vs baseline: 1.1403x; 1.0215x over previous
import functools

import jax
import jax.numpy as jnp
from jax import lax
from jax.experimental import pallas as pl
from jax.experimental.pallas import tpu as pltpu
from jax.experimental.pallas import tpu_sc as plsc

N = 10000
E = 320000
D = 128

NC = 2
NS = 16
NW = NC * NS
PER_W = E // NW
CH = 125
NCHUNK = PER_W // CH
NPH = 2
HCH = NCHUNK // NPH
NSLOT = 2
RPT = 624
TAIL = N - NS * RPT
TAIL_OFF = NS * RPT

_mesh = plsc.VectorSubcoreMesh(core_axis_name="c", subcore_axis_name="s")


@functools.partial(
    pl.kernel,
    out_type=jax.ShapeDtypeStruct((NC, N, D), jnp.float32),
    mesh=_mesh,
    scratch_types=[
        pltpu.VMEM((HCH, CH), jnp.int32),
        pltpu.VMEM((HCH, CH), jnp.int32),
        pltpu.VMEM((NSLOT * CH, D), jnp.float32),
        pltpu.VMEM_SHARED((N, D), jnp.float32),
        pltpu.SemaphoreType.DMA,
        pltpu.SemaphoreType.DMA,
    ],
)
def _sc_aggregate(m_hbm, src_hbm, dst_hbm, zero_hbm, part_hbm,
                  src_v, dst_v, rows_v, acc_sh, sem_g, sem_s):
    c = lax.axis_index("c")
    s = lax.axis_index("s")
    wid = c * NS + s
    pltpu.sync_copy(zero_hbm.at[pl.ds(s * RPT, RPT)],
                    acc_sh.at[pl.ds(s * RPT, RPT)])

    @pl.when(s == NS - 1)
    def _():
        pltpu.sync_copy(zero_hbm.at[pl.ds(TAIL_OFF, TAIL)],
                        acc_sh.at[pl.ds(TAIL_OFF, TAIL)])
    plsc.subcore_barrier()

    for ph in range(NPH):
        pltpu.sync_copy(src_hbm.at[wid, ph], src_v)
        pltpu.sync_copy(dst_hbm.at[wid, ph], dst_v)
        pltpu.async_copy(m_hbm.at[src_v.at[0]], rows_v.at[pl.ds(0, CH)],
                         sem_g)

        def body(j, carry):
            slot = lax.rem(j, NSLOT) * CH
            pltpu.make_async_copy(m_hbm.at[src_v.at[j]],
                                  rows_v.at[pl.ds(slot, CH)], sem_g).wait()

            @pl.when(j < HCH - 1)
            def _():
                nslot = lax.rem(j + 1, NSLOT) * CH

                @pl.when(j >= NSLOT - 1)
                def _():
                    pltpu.make_async_copy(
                        rows_v.at[pl.ds(nslot, CH)],
                        acc_sh.at[dst_v.at[j + 1 - NSLOT]], sem_s).wait()

                pltpu.async_copy(m_hbm.at[src_v.at[j + 1]],
                                 rows_v.at[pl.ds(nslot, CH)], sem_g)

            pltpu.async_copy(rows_v.at[pl.ds(slot, CH)],
                             acc_sh.at[dst_v.at[j]], sem_s, add=True)
            return carry

        lax.fori_loop(0, HCH, body, 0)
        for t in range(NSLOT):
            pj = HCH - NSLOT + t
            pltpu.make_async_copy(
                rows_v.at[pl.ds((pj % NSLOT) * CH, CH)],
                acc_sh.at[dst_v.at[pj]], sem_s).wait()
    plsc.subcore_barrier()
    pltpu.sync_copy(acc_sh.at[pl.ds(s * RPT, RPT)],
                    part_hbm.at[c, pl.ds(s * RPT, RPT)])

    @pl.when(s == NS - 1)
    def _():
        pltpu.sync_copy(acc_sh.at[pl.ds(TAIL_OFF, TAIL)],
                        part_hbm.at[c, pl.ds(TAIL_OFF, TAIL)])


_BLK = 5000


def _mm1_body(x_ref, w_ref, o_ref):
    o_ref[...] = jnp.dot(x_ref[...], w_ref[...],
                         preferred_element_type=jnp.float32)


def _mid_body(p_ref, b_ref, w_ref, o_ref):
    h = jnp.maximum(p_ref[0] + p_ref[1] + b_ref[...], 0.0)
    o_ref[...] = jnp.dot(h, w_ref[...], preferred_element_type=jnp.float32)


def _fin_body(p_ref, b_ref, o_ref):
    o_ref[...] = p_ref[0] + p_ref[1] + b_ref[...]


_mm1 = pl.pallas_call(
    _mm1_body,
    grid=(N // _BLK,),
    in_specs=[
        pl.BlockSpec((_BLK, D), lambda i: (i, 0)),
        pl.BlockSpec((D, D), lambda i: (0, 0)),
    ],
    out_specs=pl.BlockSpec((_BLK, D), lambda i: (i, 0)),
    out_shape=jax.ShapeDtypeStruct((N, D), jnp.float32),
)

_mid = pl.pallas_call(
    _mid_body,
    grid=(N // _BLK,),
    in_specs=[
        pl.BlockSpec((NC, _BLK, D), lambda i: (0, i, 0)),
        pl.BlockSpec((1, D), lambda i: (0, 0)),
        pl.BlockSpec((D, D), lambda i: (0, 0)),
    ],
    out_specs=pl.BlockSpec((_BLK, D), lambda i: (i, 0)),
    out_shape=jax.ShapeDtypeStruct((N, D), jnp.float32),
)

_fin = pl.pallas_call(
    _fin_body,
    grid=(N // _BLK,),
    in_specs=[
        pl.BlockSpec((NC, _BLK, D), lambda i: (0, i, 0)),
        pl.BlockSpec((1, D), lambda i: (0, 0)),
    ],
    out_specs=pl.BlockSpec((_BLK, D), lambda i: (i, 0)),
    out_shape=jax.ShapeDtypeStruct((N, D), jnp.float32),
)


def kernel(x, edge_index, W1, b1, W2, b2):
    src = edge_index[0].astype(jnp.int32).reshape(NW, NPH, HCH, CH)
    dst = edge_index[1].astype(jnp.int32).reshape(NW, NPH, HCH, CH)
    zeros = jnp.zeros((N, D), jnp.float32)
    b1r = b1.reshape(1, D)
    b2r = b2.reshape(1, D)

    m1 = _mm1(x, W1)
    p1 = _sc_aggregate(m1, src, dst, zeros)
    m2 = _mid(p1, b1r, W2)
    p2 = _sc_aggregate(m2, src, dst, zeros)
    return _fin(p2, b2r)
